# baseline (speedup 1.0000x reference)
import functools

import jax
import jax.numpy as jnp
from jax.experimental import pallas as pl
from jax.experimental.pallas import tpu as pltpu

_EMBEDDING_DIM = 32
_NUM_EMBEDDINGS = 512
_COMMITMENT_COST = 0.25
_BLOCK_M = 8192
_SUB_M = 2048


def _vq_body(x_hbm, w_ref, enc_ref, q_ref, idx_ref, loss_ref, ppl_ref,
             x_vmem, x_sem, hist_scr, sse_scr, *, n_rows):
    i = pl.program_id(0)
    bl_blk = q_ref.shape[0]

    @pl.when(i == 0)
    def _init():
        hist_scr[...] = jnp.zeros_like(hist_scr)
        sse_scr[...] = jnp.zeros_like(sse_scr)
        pltpu.make_async_copy(
            x_hbm.at[pl.ds(0, bl_blk)], x_vmem.at[0], x_sem.at[0]).start()

    slot = jax.lax.rem(i, 2)
    nxt = jax.lax.rem(i + 1, 2)

    @pl.when(i + 1 < pl.num_programs(0))
    def _prefetch():
        pltpu.make_async_copy(
            x_hbm.at[pl.ds((i + 1) * bl_blk, bl_blk)],
            x_vmem.at[nxt], x_sem.at[nxt]).start()

    pltpu.make_async_copy(
        x_hbm.at[pl.ds(i * bl_blk, bl_blk)], x_vmem.at[slot],
        x_sem.at[slot]).wait()
    x_ref = x_vmem.at[slot]

    w = w_ref[...]
    w2 = jnp.sum(w * w, axis=0, keepdims=True)
    wm2 = w * (-2.0)
    n_minor = q_ref.shape[1]
    sub_l = _SUB_M // n_minor

    idx_parts = []
    for j in range(_BLOCK_M // _SUB_M):
        x = x_ref[pl.ds(j * sub_l, sub_l), :, :].reshape(
            _SUB_M, w.shape[0])
        x2 = jnp.sum(x * x, axis=1, keepdims=True)
        xw2 = jnp.dot(x, wm2, preferred_element_type=jnp.float32)
        d = x2 + xw2 + w2

        dmin = jnp.min(d, axis=1, keepdims=True)
        iota_f = jax.lax.broadcasted_iota(jnp.int32, d.shape, 1).astype(
            jnp.float32)
        idx_f = jnp.min(jnp.where(d == dmin, iota_f, float(_NUM_EMBEDDINGS)),
                        axis=1, keepdims=True)

        onehot = (iota_f == idx_f).astype(jnp.float32)
        enc_ref[pl.ds(j * _SUB_M, _SUB_M), :] = onehot

        q = jax.lax.dot_general(onehot, w, (((1,), (1,)), ((), ())),
                                preferred_element_type=jnp.float32)
        dq = q - x
        q_ref[pl.ds(j * sub_l, sub_l), :, :] = (x + dq).reshape(
            sub_l, n_minor, x.shape[1])
        idx_parts.append(idx_f.astype(jnp.int32).reshape(sub_l, n_minor))

        ones_row = jnp.ones((1, _SUB_M), jnp.float32)
        hist_scr[...] = hist_scr[...] + jnp.dot(
            ones_row, onehot, preferred_element_type=jnp.float32)
        sse_scr[...] = sse_scr[...] + jnp.sum(dq * dq)

    idx_ref[...] = jnp.concatenate(idx_parts, axis=0)

    @pl.when(i == pl.num_programs(0) - 1)
    def _fin():
        m = sse_scr[...] / (n_rows * _EMBEDDING_DIM)
        loss_ref[...] = m + _COMMITMENT_COST * m
        p = hist_scr[...] / n_rows
        s = jnp.sum(p * jnp.log(p + 1e-10), axis=1, keepdims=True)
        ppl_ref[...] = jnp.exp(-s)


def kernel(inputs, w):
    lead_shape = inputs.shape[:-1]
    d_dim = inputs.shape[-1]
    n_rows = 1
    for s in lead_shape:
        n_rows *= s
    n_minor = lead_shape[-1]
    k_dim = w.shape[1]
    bm = _BLOCK_M
    grid = n_rows // bm
    bl = bm // n_minor

    enc, q, idx2, loss11, ppl11 = pl.pallas_call(
        functools.partial(_vq_body, n_rows=n_rows),
        grid=(grid,),
        in_specs=[
            pl.BlockSpec(memory_space=pl.ANY),
            pl.BlockSpec((d_dim, k_dim), lambda i: (0, 0)),
        ],
        out_specs=[
            pl.BlockSpec((bm, k_dim), lambda i: (i, 0)),
            pl.BlockSpec((bl, n_minor, d_dim), lambda i: (i, 0, 0)),
            pl.BlockSpec((bl, n_minor), lambda i: (i, 0)),
            pl.BlockSpec((1, 1), lambda i: (0, 0)),
            pl.BlockSpec((1, 1), lambda i: (0, 0)),
        ],
        out_shape=[
            jax.ShapeDtypeStruct((n_rows, k_dim), jnp.float32),
            jax.ShapeDtypeStruct(lead_shape + (d_dim,), jnp.float32),
            jax.ShapeDtypeStruct(lead_shape, jnp.int32),
            jax.ShapeDtypeStruct((1, 1), jnp.float32),
            jax.ShapeDtypeStruct((1, 1), jnp.float32),
        ],
        scratch_shapes=[
            pltpu.VMEM((2, bl, n_minor, d_dim), jnp.float32),
            pltpu.SemaphoreType.DMA((2,)),
            pltpu.VMEM((1, k_dim), jnp.float32),
            pltpu.VMEM((1, 1), jnp.float32),
        ],
    )(inputs, w)

    return (q, loss11[0, 0], ppl11[0, 0], enc, idx2)


# speedup vs baseline: 1.0172x; 1.0172x over previous
import functools

import jax
import jax.numpy as jnp
from jax.experimental import pallas as pl
from jax.experimental.pallas import tpu as pltpu

_EMBEDDING_DIM = 32
_NUM_EMBEDDINGS = 512
_COMMITMENT_COST = 0.25
_BLOCK_M = 8192
_SUB_M = 4096


def _vq_body(x_ref, w_ref, enc_ref, q_ref, idx_ref, loss_ref, ppl_ref,
             hist_scr, sse_scr, *, n_rows):
    i = pl.program_id(0)

    @pl.when(i == 0)
    def _init():
        hist_scr[...] = jnp.zeros_like(hist_scr)
        sse_scr[...] = jnp.zeros_like(sse_scr)

    w = w_ref[...]
    w2 = jnp.sum(w * w, axis=0, keepdims=True)
    wm2 = w * (-2.0)
    n_minor = q_ref.shape[1]
    sub_l = _SUB_M // n_minor

    iota_f = jax.lax.broadcasted_iota(
        jnp.int32, (_SUB_M, w2.shape[1]), 1).astype(jnp.float32)
    idx_parts = []
    for j in range(_BLOCK_M // _SUB_M):
        x = x_ref[pl.ds(j * sub_l, sub_l), :, :].reshape(
            _SUB_M, w.shape[0])
        x2 = jnp.sum(x * x, axis=1, keepdims=True)
        xw2 = jnp.dot(x, wm2, preferred_element_type=jnp.float32)
        d = x2 + xw2 + w2

        dmin = jnp.min(d, axis=1, keepdims=True)
        idx_f = jnp.min(jnp.where(d == dmin, iota_f, float(_NUM_EMBEDDINGS)),
                        axis=1, keepdims=True)

        onehot = (iota_f == idx_f).astype(jnp.float32)
        enc_ref[pl.ds(j * _SUB_M, _SUB_M), :] = onehot

        q = jax.lax.dot_general(onehot, w, (((1,), (1,)), ((), ())),
                                preferred_element_type=jnp.float32)
        dq = q - x
        q_ref[pl.ds(j * sub_l, sub_l), :, :] = (x + dq).reshape(
            sub_l, n_minor, x.shape[1])
        idx_parts.append(idx_f.astype(jnp.int32).reshape(sub_l, n_minor))

        ones_row = jnp.ones((1, _SUB_M), jnp.float32)
        hist_scr[...] = hist_scr[...] + jnp.dot(
            ones_row, onehot, preferred_element_type=jnp.float32)
        sse_scr[...] = sse_scr[...] + jnp.sum(dq * dq)

    idx_ref[...] = jnp.concatenate(idx_parts, axis=0)

    @pl.when(i == pl.num_programs(0) - 1)
    def _fin():
        m = sse_scr[...] / (n_rows * _EMBEDDING_DIM)
        loss_ref[...] = m + _COMMITMENT_COST * m
        p = hist_scr[...] / n_rows
        s = jnp.sum(p * jnp.log(p + 1e-10), axis=1, keepdims=True)
        ppl_ref[...] = jnp.exp(-s)


def kernel(inputs, w):
    lead_shape = inputs.shape[:-1]
    d_dim = inputs.shape[-1]
    n_rows = 1
    for s in lead_shape:
        n_rows *= s
    n_minor = lead_shape[-1]
    k_dim = w.shape[1]
    bm = _BLOCK_M
    grid = n_rows // bm
    bl = bm // n_minor

    enc, q, idx2, loss11, ppl11 = pl.pallas_call(
        functools.partial(_vq_body, n_rows=n_rows),
        grid=(grid,),
        in_specs=[
            pl.BlockSpec((bl, n_minor, d_dim), lambda i: (i, 0, 0)),
            pl.BlockSpec((d_dim, k_dim), lambda i: (0, 0)),
        ],
        out_specs=[
            pl.BlockSpec((bm, k_dim), lambda i: (i, 0)),
            pl.BlockSpec((bl, n_minor, d_dim), lambda i: (i, 0, 0)),
            pl.BlockSpec((bl, n_minor), lambda i: (i, 0)),
            pl.BlockSpec((1, 1), lambda i: (0, 0)),
            pl.BlockSpec((1, 1), lambda i: (0, 0)),
        ],
        out_shape=[
            jax.ShapeDtypeStruct((n_rows, k_dim), jnp.float32),
            jax.ShapeDtypeStruct(lead_shape + (d_dim,), jnp.float32),
            jax.ShapeDtypeStruct(lead_shape, jnp.int32),
            jax.ShapeDtypeStruct((1, 1), jnp.float32),
            jax.ShapeDtypeStruct((1, 1), jnp.float32),
        ],
        scratch_shapes=[
            pltpu.VMEM((1, k_dim), jnp.float32),
            pltpu.VMEM((1, 1), jnp.float32),
        ],
    )(inputs, w)

    return (q, loss11[0, 0], ppl11[0, 0], enc, idx2)
